# baseline (speedup 1.0000x reference)
import functools

import jax
import jax.numpy as jnp
from jax import lax
from jax.experimental import pallas as pl
from jax.experimental.pallas import tpu as pltpu
from jax.experimental.pallas import tpu_sc as plsc

F32 = jnp.float32
NC, NS, NL = 2, 16, 16
NW = NC * NS
BN = 2000


def _chunk(m):
    best = 8
    for c in range(8, 129, 8):
        if m % c == 0:
            best = c
    return best


def _mm(x, w, b):
    n, k = x.shape
    m = w.shape[1]

    def body(x_ref, w_ref, b_ref, o_ref):
        o_ref[...] = (
            jnp.dot(x_ref[...], w_ref[...], preferred_element_type=F32)
            + b_ref[...]
        )

    return pl.pallas_call(
        body,
        grid=(n // BN,),
        in_specs=[
            pl.BlockSpec((BN, k), lambda i: (i, 0)),
            pl.BlockSpec((k, m), lambda i: (0, 0)),
            pl.BlockSpec((1, m), lambda i: (0, 0)),
        ],
        out_specs=pl.BlockSpec((BN, m), lambda i: (i, 0)),
        out_shape=jax.ShapeDtypeStruct((n, m), F32),
    )(x, w, b.reshape(1, -1))


def _node_dense(h, p, wh, wp, b):
    n = h.shape[0]
    m = wh.shape[1]
    nout = m // 128

    def body(h_ref, p_ref, wh_ref, wp_ref, b_ref, *outs):
        y = (
            jnp.dot(h_ref[...], wh_ref[...], preferred_element_type=F32)
            + jnp.dot(p_ref[...], wp_ref[...], preferred_element_type=F32)
            + b_ref[...]
        )
        for j, o_ref in enumerate(outs):
            o_ref[...] = y[:, j * 128:(j + 1) * 128]

    return pl.pallas_call(
        body,
        grid=(n // BN,),
        in_specs=[
            pl.BlockSpec((BN, 128), lambda i: (i, 0)),
            pl.BlockSpec((BN, 128), lambda i: (i, 0)),
            pl.BlockSpec((128, m), lambda i: (0, 0)),
            pl.BlockSpec((128, m), lambda i: (0, 0)),
            pl.BlockSpec((1, m), lambda i: (0, 0)),
        ],
        out_specs=[pl.BlockSpec((BN, 128), lambda i: (i, 0))] * nout,
        out_shape=[jax.ShapeDtypeStruct((n, 128), F32)] * nout,
    )(h, p, wh, wp, b.reshape(1, -1))


def _edge_fuse(bd, bs, b3, vg, cg, se, te, want_enx):
    e = bd.shape[0]
    want_p = cg is not None
    nin = 5 if want_p else 4
    nout = 2 + (1 if want_p else 0) + (1 if want_enx else 0)

    def body(*refs):
        ins = refs[:nin]
        se_ref, te_ref = refs[nin], refs[nin + 1]
        outs = refs[nin + 2:]
        hat = ins[0][...] + ins[1][...] + ins[2][...]
        sig = jax.nn.sigmoid(hat)
        res = [sig, sig * ins[3][...]]
        if want_p:
            res.append(sig * ins[4][...])
        if want_enx:
            res.append(jnp.maximum(hat * se_ref[...] + te_ref[...], 0.0))
        for o_ref, val in zip(outs, res):
            o_ref[...] = val

    args = [bd, bs, b3, vg] + ([cg] if want_p else [])
    return pl.pallas_call(
        body,
        grid=(e // BN,),
        in_specs=[pl.BlockSpec((BN, 128), lambda i: (i, 0))] * nin
        + [pl.BlockSpec((1, 128), lambda i: (0, 0))] * 2,
        out_specs=[pl.BlockSpec((BN, 128), lambda i: (i, 0))] * nout,
        out_shape=[jax.ShapeDtypeStruct((e, 128), F32)] * nout,
    )(*args, se.reshape(1, -1), te.reshape(1, -1))


def _node_update(ssum, sev, sep, a1, c1, sh, th, res, stacked):
    want_p = sep is not None
    have_res = res is not None
    nseg = 3 if want_p else 2
    n = ssum.shape[1] if stacked else ssum.shape[0]
    nin = nseg + (2 if want_p else 1) + (1 if have_res else 0)

    def body(*refs):
        i = 0
        segs = []
        for _ in range(nseg):
            r_ = refs[i]; i += 1
            segs.append(r_[0] + r_[1] if stacked else r_[...])
        a1_ref = refs[i]; i += 1
        if want_p:
            c1_ref = refs[i]; i += 1
        if have_res:
            res_ref = refs[i]; i += 1
        sh_ref = refs[i]; i += 1
        th_ref = refs[i]; i += 1
        outs = refs[i:]
        r = 1.0 / (segs[0] + 1e-6)
        h = jnp.maximum(
            (a1_ref[...] + r * segs[1]) * sh_ref[...] + th_ref[...], 0.0
        )
        if have_res:
            h = h + res_ref[...]
        outs[0][...] = h
        if want_p:
            outs[1][...] = jnp.tanh(c1_ref[...] + r * segs[2])

    args = [ssum, sev] + ([sep] if want_p else []) + [a1] \
        + ([c1] if want_p else [])
    if have_res:
        args += [res]
    nout = 2 if want_p else 1
    seg_spec = (
        pl.BlockSpec((NC, BN, 128), lambda i: (0, i, 0))
        if stacked else pl.BlockSpec((BN, 128), lambda i: (i, 0))
    )
    return pl.pallas_call(
        body,
        grid=(n // BN,),
        in_specs=[seg_spec] * nseg
        + [pl.BlockSpec((BN, 128), lambda i: (i, 0))] * (nin - nseg)
        + [pl.BlockSpec((1, 128), lambda i: (0, 0))] * 2,
        out_specs=[pl.BlockSpec((BN, 128), lambda i: (i, 0))] * nout,
        out_shape=[jax.ShapeDtypeStruct((n, 128), F32)] * nout,
    )(*args, sh.reshape(1, -1), th.reshape(1, -1))


def _embed_p(tok, pos, res_emb, atom_emb, ppw, ppb, g, b):
    n = tok.shape[0]
    kr = res_emb.shape[0]
    ka = atom_emb.shape[0]

    def body(tok_ref, pos_ref, re_ref, ae_ref, ppw_ref, ppb_ref, g_ref,
             b_ref, h_ref, pp_ref):
        tr = tok_ref[:, 0:1]
        ta = tok_ref[:, 1:2]
        ohr = (tr == lax.broadcasted_iota(jnp.int32, (1, kr), 1)).astype(F32)
        oha = (ta == lax.broadcasted_iota(jnp.int32, (1, ka), 1)).astype(F32)
        hr = jnp.dot(ohr, re_ref[...], preferred_element_type=F32,
                     precision=lax.Precision.HIGHEST)
        ha = jnp.dot(oha, ae_ref[...], preferred_element_type=F32,
                     precision=lax.Precision.HIGHEST)
        x = jnp.concatenate([hr, ha], axis=1)
        mu = jnp.mean(x, axis=-1, keepdims=True)
        var = jnp.mean((x - mu) ** 2, axis=-1, keepdims=True)
        h_ref[...] = (x - mu) / jnp.sqrt(var + 1e-5) * g_ref[...] + b_ref[...]
        pp_ref[...] = (
            jnp.dot(pos_ref[...], ppw_ref[...], preferred_element_type=F32)
            + ppb_ref[...]
        )

    return pl.pallas_call(
        body,
        grid=(n // BN,),
        in_specs=[
            pl.BlockSpec((BN, 2), lambda i: (i, 0)),
            pl.BlockSpec((BN, 16), lambda i: (i, 0)),
            pl.BlockSpec((kr, 64), lambda i: (0, 0)),
            pl.BlockSpec((ka, 64), lambda i: (0, 0)),
            pl.BlockSpec((16, 128), lambda i: (0, 0)),
            pl.BlockSpec((1, 128), lambda i: (0, 0)),
            pl.BlockSpec((1, 128), lambda i: (0, 0)),
            pl.BlockSpec((1, 128), lambda i: (0, 0)),
        ],
        out_specs=[pl.BlockSpec((BN, 128), lambda i: (i, 0))] * 2,
        out_shape=[jax.ShapeDtypeStruct((n, 128), F32)] * 2,
    )(tok, pos, res_emb, atom_emb, ppw, ppb.reshape(1, -1),
      g.reshape(1, -1), b.reshape(1, -1))


def _embed_l(feat, pos, lnw, lnb, lpw, lpb, g, b):
    n = feat.shape[0]

    def body(f_ref, pos_ref, lnw_ref, lnb_ref, lpw_ref, lpb_ref, g_ref,
             b_ref, h_ref, pp_ref):
        x = (
            jnp.dot(f_ref[...], lnw_ref[...], preferred_element_type=F32)
            + lnb_ref[...]
        )
        mu = jnp.mean(x, axis=-1, keepdims=True)
        var = jnp.mean((x - mu) ** 2, axis=-1, keepdims=True)
        h_ref[...] = (x - mu) / jnp.sqrt(var + 1e-5) * g_ref[...] + b_ref[...]
        pp_ref[...] = (
            jnp.dot(pos_ref[...], lpw_ref[...], preferred_element_type=F32)
            + lpb_ref[...]
        )

    return pl.pallas_call(
        body,
        grid=(n // BN,),
        in_specs=[
            pl.BlockSpec((BN, 128), lambda i: (i, 0)),
            pl.BlockSpec((BN, 16), lambda i: (i, 0)),
            pl.BlockSpec((128, 128), lambda i: (0, 0)),
            pl.BlockSpec((1, 128), lambda i: (0, 0)),
            pl.BlockSpec((16, 128), lambda i: (0, 0)),
            pl.BlockSpec((1, 128), lambda i: (0, 0)),
            pl.BlockSpec((1, 128), lambda i: (0, 0)),
            pl.BlockSpec((1, 128), lambda i: (0, 0)),
        ],
        out_specs=[pl.BlockSpec((BN, 128), lambda i: (i, 0))] * 2,
        out_shape=[jax.ShapeDtypeStruct((n, 128), F32)] * 2,
    )(feat, pos, lnw, lnb.reshape(1, -1), lpw, lpb.reshape(1, -1),
      g.reshape(1, -1), b.reshape(1, -1))


def _final(hc, w1, b1, sm, tm, w2, b2):
    n = hc.shape[0]

    def body(x_ref, w1_ref, b1_ref, sm_ref, tm_ref, w2_ref, b2_ref, o_ref):
        s = jnp.sum(x_ref[...], axis=0, keepdims=True)
        y = (
            jnp.dot(s, w1_ref[...], preferred_element_type=F32) + b1_ref[...]
        ) * sm_ref[...] + tm_ref[...]
        y = jnp.where(y > 0.0, y, jnp.exp(y) - 1.0)
        o_ref[...] = (
            jnp.dot(y, w2_ref[...], preferred_element_type=F32) + b2_ref[...]
        )

    return pl.pallas_call(
        body,
        grid=(1,),
        in_specs=[
            pl.BlockSpec((n, 128), lambda i: (0, 0)),
            pl.BlockSpec((128, 128), lambda i: (0, 0)),
            pl.BlockSpec((1, 128), lambda i: (0, 0)),
            pl.BlockSpec((1, 128), lambda i: (0, 0)),
            pl.BlockSpec((1, 128), lambda i: (0, 0)),
            pl.BlockSpec((128, 1), lambda i: (0, 0)),
            pl.BlockSpec((1, 1), lambda i: (0, 0)),
        ],
        out_specs=pl.BlockSpec((1, 1), lambda i: (0, 0)),
        out_shape=jax.ShapeDtypeStruct((1, 1), F32),
    )(hc, w1, b1.reshape(1, -1), sm.reshape(1, -1), tm.reshape(1, -1),
      w2, b2.reshape(1, -1))


def _sc_gather(dst, src, t_dst, tables_src):
    e = dst.shape[0]
    ew = e // NW
    ch = _chunk(ew)
    nit = ew // ch
    ksrc = len(tables_src)
    k = 1 + ksrc
    mesh = plsc.VectorSubcoreMesh(core_axis_name="c", subcore_axis_name="s")

    def body(*refs):
        dst_h, src_h = refs[0], refs[1]
        tbls = refs[2:2 + k]
        outs = refs[2 + k:2 + 2 * k]
        scr = refs[2 + 2 * k:]
        idxd, idxs = scr[0], scr[1]
        bufs = scr[2:2 + k]
        sems = scr[2 + k:]
        wid = lax.axis_index("s") * NC + lax.axis_index("c")
        base = wid * ew

        def step(i, carry):
            off = base + i * ch
            pltpu.sync_copy(dst_h.at[pl.ds(off, ch)], idxd)
            pltpu.sync_copy(src_h.at[pl.ds(off, ch)], idxs)
            cps = []
            for j in range(k):
                idx = idxd if j == 0 else idxs
                cps.append(pltpu.async_copy(tbls[j].at[idx], bufs[j], sems[j]))
            for j in range(k):
                cps[j].wait()
                pltpu.sync_copy(bufs[j], outs[j].at[pl.ds(off, ch)])
            return carry

        lax.fori_loop(0, nit, step, 0)

    fn = pl.kernel(
        body,
        out_type=tuple(jax.ShapeDtypeStruct((e, 128), F32) for _ in range(k)),
        mesh=mesh,
        scratch_types=(
            [pltpu.VMEM((ch,), jnp.int32)] * 2
            + [pltpu.VMEM((ch, 128), F32) for _ in range(k)]
            + [pltpu.SemaphoreType.DMA for _ in range(k)]
        ),
    )
    return fn(dst, src, t_dst, *tables_src)


def _chunk_cap(m, cap):
    best = 8
    for c in range(8, cap + 1, 8):
        if m % c == 0:
            best = c
    return best


def _sc_k1(dst, src, b1h, b2h, b3, se, te, n, zrows, want_enx):
    e = dst.shape[0]
    n_pad = 128 * ((n + 127) // 128)
    eh = e // NC
    et = eh // NS
    ch = _chunk_cap(et, 40)
    nit = et // ch
    mesh = plsc.VectorSubcoreMesh(core_axis_name="c", subcore_axis_name="s")

    def body(*refs):
        (dst_h, src_h, b1_h, b2_h, b3_h, se_h, te_h, z_h) = refs[:8]
        outs = refs[8:8 + (3 if want_enx else 2)]
        scr = refs[8 + (3 if want_enx else 2):]
        ssum_h, sig_h = outs[0], outs[1]
        enx_h = outs[2] if want_enx else None
        idxd, idxs = scr[0], scr[1]
        bufd, bufs, bufb, sigb = scr[2], scr[3], scr[4], scr[5]
        i = 6
        enxb = scr[i] if want_enx else None
        i += 1 if want_enx else 0
        sev, tev = scr[i], scr[i + 1]
        acc = scr[i + 2]
        semd, sems = scr[i + 3], scr[i + 4]
        cid = lax.axis_index("c")
        sid = lax.axis_index("s")

        rpt = n_pad // NS
        r0 = sid * rpt
        off = 0
        while off < rpt:
            sz = min(128, rpt - off)
            pltpu.sync_copy(z_h.at[pl.ds(0, sz)], acc.at[pl.ds(r0 + off, sz)])
            off += sz
        pltpu.sync_copy(se_h, sev)
        pltpu.sync_copy(te_h, tev)
        plsc.subcore_barrier()

        sejs = [sev[pl.ds(j * 16, 16)] for j in range(8)]
        tejs = [tev[pl.ds(j * 16, 16)] for j in range(8)]

        tbase = cid * eh + sid * et

        def step(it, carry):
            o = tbase + it * ch
            pltpu.sync_copy(dst_h.at[pl.ds(o, ch)], idxd)
            pltpu.sync_copy(src_h.at[pl.ds(o, ch)], idxs)
            cpd = pltpu.async_copy(b1_h.at[idxd], bufd, semd)
            cps = pltpu.async_copy(b2_h.at[idxs], bufs, sems)
            pltpu.sync_copy(b3_h.at[pl.ds(o, ch)], bufb)
            cpd.wait()
            cps.wait()
            for r in range(ch):
                for j in range(8):
                    s16 = pl.ds(j * 16, 16)
                    hat = bufd[r, s16] + bufs[r, s16] + bufb[r, s16]
                    sg = 1.0 / (1.0 + jnp.exp(-hat))
                    sigb[r, s16] = sg
                    if want_enx:
                        enxb[r, s16] = jnp.maximum(
                            hat * sejs[j] + tejs[j], 0.0)
            pltpu.sync_copy(sigb, sig_h.at[pl.ds(o, ch)])
            if want_enx:
                pltpu.sync_copy(enxb, enx_h.at[pl.ds(o, ch)])
            pltpu.sync_copy(sigb, acc.at[idxd], add=True)
            return carry

        lax.fori_loop(0, nit, step, 0)
        plsc.subcore_barrier()

        o = 0
        while o < rpt:
            sz = min(128, rpt - o)
            pltpu.sync_copy(
                acc.at[pl.ds(r0 + o, sz)],
                ssum_h.at[cid, pl.ds(r0 + o, sz)],
            )
            o += sz

    out_type = [
        jax.ShapeDtypeStruct((NC, n_pad, 128), F32),
        jax.ShapeDtypeStruct((e, 128), F32),
    ]
    if want_enx:
        out_type.append(jax.ShapeDtypeStruct((e, 128), F32))
    scratch = [
        pltpu.VMEM((ch,), jnp.int32),
        pltpu.VMEM((ch,), jnp.int32),
        pltpu.VMEM((ch, 128), F32),
        pltpu.VMEM((ch, 128), F32),
        pltpu.VMEM((ch, 128), F32),
        pltpu.VMEM((ch, 128), F32),
    ]
    if want_enx:
        scratch.append(pltpu.VMEM((ch, 128), F32))
    scratch += [
        pltpu.VMEM((128,), F32),
        pltpu.VMEM((128,), F32),
        pltpu.VMEM_SHARED((n_pad, 128), F32),
        pltpu.SemaphoreType.DMA,
        pltpu.SemaphoreType.DMA,
    ]
    fn = pl.kernel(
        body,
        out_type=tuple(out_type),
        mesh=mesh,
        scratch_types=tuple(scratch),
    )
    res = fn(dst, src, b1h, b2h, b3, se, te, zrows)
    ssum = res[0][:, :n] if n_pad != n else res[0]
    return (ssum,) + tuple(res[1:])


def _sc_k2(dst, src, table, sig, n, zrows):
    e = dst.shape[0]
    n_pad = 128 * ((n + 127) // 128)
    eh = e // NC
    et = eh // NS
    ch = _chunk_cap(et, 80)
    nit = et // ch
    mesh = plsc.VectorSubcoreMesh(core_axis_name="c", subcore_axis_name="s")

    def body(dst_h, src_h, tab_h, sig_h, z_h, out_h,
             idxd, idxs, bufv, bufsg, acc, semv):
        cid = lax.axis_index("c")
        sid = lax.axis_index("s")

        rpt = n_pad // NS
        r0 = sid * rpt
        off = 0
        while off < rpt:
            sz = min(128, rpt - off)
            pltpu.sync_copy(z_h.at[pl.ds(0, sz)], acc.at[pl.ds(r0 + off, sz)])
            off += sz
        plsc.subcore_barrier()

        tbase = cid * eh + sid * et

        def step(it, carry):
            o = tbase + it * ch
            pltpu.sync_copy(dst_h.at[pl.ds(o, ch)], idxd)
            pltpu.sync_copy(src_h.at[pl.ds(o, ch)], idxs)
            cpv = pltpu.async_copy(tab_h.at[idxs], bufv, semv)
            pltpu.sync_copy(sig_h.at[pl.ds(o, ch)], bufsg)
            cpv.wait()
            for r in range(ch):
                for j in range(8):
                    s16 = pl.ds(j * 16, 16)
                    bufsg[r, s16] = bufsg[r, s16] * bufv[r, s16]
            pltpu.sync_copy(bufsg, acc.at[idxd], add=True)
            return carry

        lax.fori_loop(0, nit, step, 0)
        plsc.subcore_barrier()

        o = 0
        while o < rpt:
            sz = min(128, rpt - o)
            pltpu.sync_copy(
                acc.at[pl.ds(r0 + o, sz)],
                out_h.at[cid, pl.ds(r0 + o, sz)],
            )
            o += sz

    fn = pl.kernel(
        body,
        out_type=jax.ShapeDtypeStruct((NC, n_pad, 128), F32),
        mesh=mesh,
        scratch_types=(
            pltpu.VMEM((ch,), jnp.int32),
            pltpu.VMEM((ch,), jnp.int32),
            pltpu.VMEM((ch, 128), F32),
            pltpu.VMEM((ch, 128), F32),
            pltpu.VMEM_SHARED((n_pad, 128), F32),
            pltpu.SemaphoreType.DMA,
        ),
    )
    out = fn(dst, src, table, sig, zrows)
    return out[:, :n] if n_pad != n else out


def _sc_scatter_es(vals, dst, n, zrows):
    e = vals.shape[0]
    n_pad = 128 * ((n + 127) // 128)
    eh = e // NC
    et = eh // NS
    ch = _chunk(et)
    nit = et // ch
    mesh = plsc.VectorSubcoreMesh(core_axis_name="c", subcore_axis_name="s")

    def body(vals_h, dst_h, z_h, out_h, idxv, buf, acc):
        cid = lax.axis_index("c")
        sid = lax.axis_index("s")

        rpt = n_pad // NS
        r0 = sid * rpt
        off = 0
        while off < rpt:
            sz = min(128, rpt - off)
            pltpu.sync_copy(z_h.at[pl.ds(0, sz)], acc.at[pl.ds(r0 + off, sz)])
            off += sz
        plsc.subcore_barrier()

        tbase = cid * eh + sid * et

        def step(i, carry):
            o = tbase + i * ch
            pltpu.sync_copy(dst_h.at[pl.ds(o, ch)], idxv)
            pltpu.sync_copy(vals_h.at[pl.ds(o, ch)], buf)
            pltpu.sync_copy(buf, acc.at[idxv], add=True)
            return carry

        lax.fori_loop(0, nit, step, 0)
        plsc.subcore_barrier()

        o = 0
        while o < rpt:
            sz = min(128, rpt - o)
            pltpu.sync_copy(
                acc.at[pl.ds(r0 + o, sz)],
                out_h.at[cid, pl.ds(r0 + o, sz)],
            )
            o += sz

    fn = pl.kernel(
        body,
        out_type=jax.ShapeDtypeStruct((NC, n_pad, 128), F32),
        mesh=mesh,
        scratch_types=(
            pltpu.VMEM((ch,), jnp.int32),
            pltpu.VMEM((ch, 128), F32),
            pltpu.VMEM_SHARED((n_pad, 128), F32),
        ),
    )
    out = fn(vals, dst, zrows)
    return out[:, :n] if n_pad != n else out


def _sc_scatter(vals, dst, n, zrows):
    e = vals.shape[0]
    n_pad = 2048 * ((n + 2047) // 2048)
    nrm = n_pad // 2
    rows = nrm + 128
    junk = nrm
    et = e // NS
    ch = _chunk(et)
    nit = et // ch
    mesh = plsc.VectorSubcoreMesh(core_axis_name="c", subcore_axis_name="s")

    def body(vals_h, dst_h, z_h, out_h, idxv, locv, buf, acc):
        cid = lax.axis_index("c")
        sid = lax.axis_index("s")
        rbase = (cid * nrm).astype(jnp.int32)

        rpt_i = rows // NS
        r0 = sid * rpt_i
        off = 0
        while off < rpt_i:
            sz = min(128, rpt_i - off)
            pltpu.sync_copy(z_h.at[pl.ds(0, sz)], acc.at[pl.ds(r0 + off, sz)])
            off += sz
        plsc.subcore_barrier()

        tbase = sid * et

        def step(i, carry):
            o = tbase + i * ch
            pltpu.sync_copy(dst_h.at[pl.ds(o, ch)], idxv)
            pltpu.sync_copy(vals_h.at[pl.ds(o, ch)], buf)
            for j in range(ch // 16):
                t = idxv[pl.ds(j * 16, 16)] - rbase
                ok = (t >= 0) & (t < nrm)
                locv[pl.ds(j * 16, 16)] = jnp.where(ok, t, junk)
            pltpu.sync_copy(buf, acc.at[locv], add=True)
            return carry

        lax.fori_loop(0, nit, step, 0)
        plsc.subcore_barrier()

        rpt = nrm // NS
        rr0 = sid * rpt
        o = 0
        while o < rpt:
            sz = min(128, rpt - o)
            pltpu.sync_copy(
                acc.at[pl.ds(rr0 + o, sz)],
                out_h.at[pl.ds(rbase + rr0 + o, sz)],
            )
            o += sz

    fn = pl.kernel(
        body,
        out_type=jax.ShapeDtypeStruct((n_pad, 128), F32),
        mesh=mesh,
        scratch_types=(
            pltpu.VMEM((ch,), jnp.int32),
            pltpu.VMEM((ch,), jnp.int32),
            pltpu.VMEM((ch, 128), F32),
            pltpu.VMEM_SHARED((rows, 128), F32),
        ),
    )
    out = fn(vals, dst, zrows)
    return out[:n] if n_pad != n else out


def _fold_bn(g, b, m, v):
    s = g / jnp.sqrt(v + 1e-5)
    return s, b - m * s


def _gated(lp, src, dst, h, p, b3, n, zrows, want_p, want_enx, res=None):
    if want_p:
        wh = jnp.concatenate(
            [lp["B1_W"], lp["B2_W"], lp["A1_W"][:128], lp["A2_W"][:128],
             jnp.zeros((128, 256), F32)], axis=1)
        wp = jnp.concatenate(
            [jnp.zeros((128, 256), F32), lp["A1_W"][128:], lp["A2_W"][128:],
             lp["C1_W"], lp["C2_W"]], axis=1)
        bb = jnp.concatenate(
            [lp["B1_b"], lp["B2_b"], lp["A1_b"], lp["A2_b"],
             lp["C1_b"], lp["C2_b"]])
        b1h, b2h, a1, v, c1, cp = _node_dense(h, p, wh, wp, bb)
    else:
        wh = jnp.concatenate(
            [lp["B1_W"], lp["B2_W"], lp["A1_W"][:128], lp["A2_W"][:128]],
            axis=1)
        wp = jnp.concatenate(
            [jnp.zeros((128, 256), F32), lp["A1_W"][128:], lp["A2_W"][128:]],
            axis=1)
        bb = jnp.concatenate(
            [lp["B1_b"], lp["B2_b"], lp["A1_b"], lp["A2_b"]])
        b1h, b2h, a1, v = _node_dense(h, p, wh, wp, bb)
        c1 = cp = None

    se, te = _fold_bn(lp["bne_g"], lp["bne_b"], lp["bne_m"], lp["bne_v"])
    sh, th = _fold_bn(lp["bnh_g"], lp["bnh_b"], lp["bnh_m"], lp["bnh_v"])

    if want_p:
        k1 = _sc_k1(dst, src, b1h, b2h, b3, se, te, n, zrows, want_enx)
        ssum, sig = k1[0], k1[1]
        enx = k1[2] if want_enx else None
        sev = _sc_k2(dst, src, v, sig, n, zrows)
        sep = _sc_k2(dst, src, cp, sig, n, zrows)
        nu = _node_update(ssum, sev, sep, a1, c1, sh, th, res, True)
        return nu[0], nu[1], enx

    tables = [b2h, v]
    gathered = _sc_gather(dst, src, b1h, tables)
    bd, bs, vg = gathered[0], gathered[1], gathered[2]

    ef = _edge_fuse(bd, bs, b3, vg, None, se, te, want_enx)
    sig, ev = ef[0], ef[1]
    enx = ef[2] if want_enx else None

    stacked = n <= 16384
    scat = _sc_scatter_es if stacked else _sc_scatter
    ssum = scat(sig, dst, n, zrows)
    sev = scat(ev, dst, n, zrows)

    nu = _node_update(ssum, sev, None, a1, c1, sh, th, res, stacked)
    return nu[0], None, enx


def kernel(gp_token_res, gp_token_atom, gp_pos_enc, gp_dist, gp_edge_index,
           gl_feat, gl_pos_enc, gl_edge_feat, gl_edge_index,
           gc_dist, gc_edge_index, params):
    pr = params
    n_p = gp_token_res.shape[0]
    n_l = gl_feat.shape[0]
    n_c = n_p + n_l

    zrows = jnp.zeros((128, 128), F32)

    tok = jnp.stack(
        [gp_token_res.astype(jnp.int32), gp_token_atom.astype(jnp.int32)],
        axis=1)
    res_pad = jnp.pad(pr["res_emb"], ((0, 2), (0, 0)))
    atom_pad = jnp.pad(pr["atom_emb"], ((0, 1), (0, 0)))
    hp, pp = _embed_p(tok, gp_pos_enc, res_pad, atom_pad,
                      pr["pp_W"], pr["pp_b"], pr["pnorm_g"], pr["pnorm_b"])
    hl, pl_ = _embed_l(gl_feat, gl_pos_enc, pr["ln_W"], pr["ln_b"],
                       pr["lp_W"], pr["lp_b"], pr["lnorm_g"], pr["lnorm_b"])
    hp_raw, hl_raw = hp, hl
    res_c = jnp.concatenate([hp_raw, hl_raw], axis=0)

    ps, pd = gp_edge_index[0], gp_edge_index[1]
    ls, ld = gl_edge_index[0], gl_edge_index[1]
    cs, cd = gc_edge_index[0], gc_edge_index[1]

    xp = jnp.pad(gp_dist, ((0, 0), (0, 1)))
    xc = jnp.pad(gc_dist, ((0, 0), (0, 1)))
    pe_w = jnp.pad(pr["pe_W"], ((0, 1), (0, 0)))
    ce_w = jnp.pad(pr["ce_W"], ((0, 1), (0, 0)))

    def b3_first(x, ew, eb, blk):
        return _mm(_mm(x, ew, eb), blk["B3_W"], blk["B3_b"])

    enx_p = enx_l = enx_c = None
    hc = None
    for i in range(3):
        bp, bl, bc = pr["pblock"][i], pr["lblock"][i], pr["cblock"][i]
        last = i == 2

        if i == 0:
            b3p = b3_first(xp, pe_w, pr["pe_b"], bp)
            b3l = b3_first(gl_edge_feat, pr["le_W"], pr["le_b"], bl)
        else:
            b3p = _mm(enx_p, bp["B3_W"], bp["B3_b"])
            b3l = _mm(enx_l, bl["B3_W"], bl["B3_b"])
        hp, pp, enx_p = _gated(bp, ps, pd, hp, pp, b3p, n_p, zrows,
                               want_p=True, want_enx=not last)
        hl, pl_, enx_l = _gated(bl, ls, ld, hl, pl_, b3l, n_l, zrows,
                                want_p=True, want_enx=not last)

        hcat = jnp.concatenate([hp, hl], axis=0)
        pcat = jnp.concatenate([pp, pl_], axis=0)
        if i == 0:
            b3c = b3_first(xc, ce_w, pr["ce_b"], bc)
        else:
            b3c = _mm(enx_c, bc["B3_W"], bc["B3_b"])
        hc, _, enx_c = _gated(bc, cs, cd, hcat, pcat, b3c, n_c, zrows,
                              want_p=False, want_enx=not last,
                              res=None if last else res_c)
        if not last:
            hp = hc[:n_p]
            hl = hc[n_p:]

    sm, tm = _fold_bn(pr["mbn_g"], pr["mbn_b"], pr["mbn_m"], pr["mbn_v"])
    return _final(hc, pr["mlp1_W"], pr["mlp1_b"], sm, tm,
                  pr["mlp2_W"], pr["mlp2_b"])


# speedup vs baseline: 1.3259x; 1.3259x over previous
import functools

import jax
import jax.numpy as jnp
from jax import lax
from jax.experimental import pallas as pl
from jax.experimental.pallas import tpu as pltpu
from jax.experimental.pallas import tpu_sc as plsc

F32 = jnp.float32
NC, NS, NL = 2, 16, 16
NW = NC * NS
BN = 2000


def _chunk(m):
    best = 8
    for c in range(8, 129, 8):
        if m % c == 0:
            best = c
    return best


def _mm(x, w, b):
    n, k = x.shape
    m = w.shape[1]

    def body(x_ref, w_ref, b_ref, o_ref):
        o_ref[...] = (
            jnp.dot(x_ref[...], w_ref[...], preferred_element_type=F32)
            + b_ref[...]
        )

    return pl.pallas_call(
        body,
        grid=(n // BN,),
        in_specs=[
            pl.BlockSpec((BN, k), lambda i: (i, 0)),
            pl.BlockSpec((k, m), lambda i: (0, 0)),
            pl.BlockSpec((1, m), lambda i: (0, 0)),
        ],
        out_specs=pl.BlockSpec((BN, m), lambda i: (i, 0)),
        out_shape=jax.ShapeDtypeStruct((n, m), F32),
    )(x, w, b.reshape(1, -1))


def _node_dense(h, p, wh, wp, b):
    n = h.shape[0]
    m = wh.shape[1]
    nout = m // 128

    def body(h_ref, p_ref, wh_ref, wp_ref, b_ref, *outs):
        y = (
            jnp.dot(h_ref[...], wh_ref[...], preferred_element_type=F32)
            + jnp.dot(p_ref[...], wp_ref[...], preferred_element_type=F32)
            + b_ref[...]
        )
        for j, o_ref in enumerate(outs):
            o_ref[...] = y[:, j * 128:(j + 1) * 128]

    return pl.pallas_call(
        body,
        grid=(n // BN,),
        in_specs=[
            pl.BlockSpec((BN, 128), lambda i: (i, 0)),
            pl.BlockSpec((BN, 128), lambda i: (i, 0)),
            pl.BlockSpec((128, m), lambda i: (0, 0)),
            pl.BlockSpec((128, m), lambda i: (0, 0)),
            pl.BlockSpec((1, m), lambda i: (0, 0)),
        ],
        out_specs=[pl.BlockSpec((BN, 128), lambda i: (i, 0))] * nout,
        out_shape=[jax.ShapeDtypeStruct((n, 128), F32)] * nout,
    )(h, p, wh, wp, b.reshape(1, -1))


def _edge_fuse(bd, bs, b3, vg, cg, se, te, want_enx):
    e = bd.shape[0]
    want_v = vg is not None
    want_p = cg is not None
    nin = 3 + (1 if want_v else 0) + (1 if want_p else 0)
    nout = 1 + (1 if want_v else 0) + (1 if want_p else 0) \
        + (1 if want_enx else 0)

    def body(*refs):
        ins = refs[:nin]
        se_ref, te_ref = refs[nin], refs[nin + 1]
        outs = refs[nin + 2:]
        hat = ins[0][...] + ins[1][...] + ins[2][...]
        sig = jax.nn.sigmoid(hat)
        res = [sig]
        if want_v:
            res.append(sig * ins[3][...])
        if want_p:
            res.append(sig * ins[4][...])
        if want_enx:
            res.append(jnp.maximum(hat * se_ref[...] + te_ref[...], 0.0))
        for o_ref, val in zip(outs, res):
            o_ref[...] = val

    args = [bd, bs, b3] + ([vg] if want_v else []) + ([cg] if want_p else [])
    return pl.pallas_call(
        body,
        grid=(e // BN,),
        in_specs=[pl.BlockSpec((BN, 128), lambda i: (i, 0))] * nin
        + [pl.BlockSpec((1, 128), lambda i: (0, 0))] * 2,
        out_specs=[pl.BlockSpec((BN, 128), lambda i: (i, 0))] * nout,
        out_shape=[jax.ShapeDtypeStruct((e, 128), F32)] * nout,
    )(*args, se.reshape(1, -1), te.reshape(1, -1))


def _node_update(ssum, sev, sep, a1, c1, sh, th, res, stacked):
    want_p = sep is not None
    have_res = res is not None
    nseg = 3 if want_p else 2
    n = ssum.shape[1] if stacked else ssum.shape[0]
    nin = nseg + (2 if want_p else 1) + (1 if have_res else 0)

    def body(*refs):
        i = 0
        segs = []
        for _ in range(nseg):
            r_ = refs[i]; i += 1
            segs.append(r_[0] + r_[1] if stacked else r_[...])
        a1_ref = refs[i]; i += 1
        if want_p:
            c1_ref = refs[i]; i += 1
        if have_res:
            res_ref = refs[i]; i += 1
        sh_ref = refs[i]; i += 1
        th_ref = refs[i]; i += 1
        outs = refs[i:]
        r = 1.0 / (segs[0] + 1e-6)
        h = jnp.maximum(
            (a1_ref[...] + r * segs[1]) * sh_ref[...] + th_ref[...], 0.0
        )
        if have_res:
            h = h + res_ref[...]
        outs[0][...] = h
        if want_p:
            outs[1][...] = jnp.tanh(c1_ref[...] + r * segs[2])

    args = [ssum, sev] + ([sep] if want_p else []) + [a1] \
        + ([c1] if want_p else [])
    if have_res:
        args += [res]
    nout = 2 if want_p else 1
    seg_spec = (
        pl.BlockSpec((NC, BN, 128), lambda i: (0, i, 0))
        if stacked else pl.BlockSpec((BN, 128), lambda i: (i, 0))
    )
    return pl.pallas_call(
        body,
        grid=(n // BN,),
        in_specs=[seg_spec] * nseg
        + [pl.BlockSpec((BN, 128), lambda i: (i, 0))] * (nin - nseg)
        + [pl.BlockSpec((1, 128), lambda i: (0, 0))] * 2,
        out_specs=[pl.BlockSpec((BN, 128), lambda i: (i, 0))] * nout,
        out_shape=[jax.ShapeDtypeStruct((n, 128), F32)] * nout,
    )(*args, sh.reshape(1, -1), th.reshape(1, -1))


def _embed_p(tok, pos, res_emb, atom_emb, ppw, ppb, g, b):
    n = tok.shape[0]
    kr = res_emb.shape[0]
    ka = atom_emb.shape[0]

    def body(tok_ref, pos_ref, re_ref, ae_ref, ppw_ref, ppb_ref, g_ref,
             b_ref, h_ref, pp_ref):
        tr = tok_ref[:, 0:1]
        ta = tok_ref[:, 1:2]
        ohr = (tr == lax.broadcasted_iota(jnp.int32, (1, kr), 1)).astype(F32)
        oha = (ta == lax.broadcasted_iota(jnp.int32, (1, ka), 1)).astype(F32)
        hr = jnp.dot(ohr, re_ref[...], preferred_element_type=F32,
                     precision=lax.Precision.HIGHEST)
        ha = jnp.dot(oha, ae_ref[...], preferred_element_type=F32,
                     precision=lax.Precision.HIGHEST)
        x = jnp.concatenate([hr, ha], axis=1)
        mu = jnp.mean(x, axis=-1, keepdims=True)
        var = jnp.mean((x - mu) ** 2, axis=-1, keepdims=True)
        h_ref[...] = (x - mu) / jnp.sqrt(var + 1e-5) * g_ref[...] + b_ref[...]
        pp_ref[...] = (
            jnp.dot(pos_ref[...], ppw_ref[...], preferred_element_type=F32)
            + ppb_ref[...]
        )

    return pl.pallas_call(
        body,
        grid=(n // BN,),
        in_specs=[
            pl.BlockSpec((BN, 2), lambda i: (i, 0)),
            pl.BlockSpec((BN, 16), lambda i: (i, 0)),
            pl.BlockSpec((kr, 64), lambda i: (0, 0)),
            pl.BlockSpec((ka, 64), lambda i: (0, 0)),
            pl.BlockSpec((16, 128), lambda i: (0, 0)),
            pl.BlockSpec((1, 128), lambda i: (0, 0)),
            pl.BlockSpec((1, 128), lambda i: (0, 0)),
            pl.BlockSpec((1, 128), lambda i: (0, 0)),
        ],
        out_specs=[pl.BlockSpec((BN, 128), lambda i: (i, 0))] * 2,
        out_shape=[jax.ShapeDtypeStruct((n, 128), F32)] * 2,
    )(tok, pos, res_emb, atom_emb, ppw, ppb.reshape(1, -1),
      g.reshape(1, -1), b.reshape(1, -1))


def _embed_l(feat, pos, lnw, lnb, lpw, lpb, g, b):
    n = feat.shape[0]

    def body(f_ref, pos_ref, lnw_ref, lnb_ref, lpw_ref, lpb_ref, g_ref,
             b_ref, h_ref, pp_ref):
        x = (
            jnp.dot(f_ref[...], lnw_ref[...], preferred_element_type=F32)
            + lnb_ref[...]
        )
        mu = jnp.mean(x, axis=-1, keepdims=True)
        var = jnp.mean((x - mu) ** 2, axis=-1, keepdims=True)
        h_ref[...] = (x - mu) / jnp.sqrt(var + 1e-5) * g_ref[...] + b_ref[...]
        pp_ref[...] = (
            jnp.dot(pos_ref[...], lpw_ref[...], preferred_element_type=F32)
            + lpb_ref[...]
        )

    return pl.pallas_call(
        body,
        grid=(n // BN,),
        in_specs=[
            pl.BlockSpec((BN, 128), lambda i: (i, 0)),
            pl.BlockSpec((BN, 16), lambda i: (i, 0)),
            pl.BlockSpec((128, 128), lambda i: (0, 0)),
            pl.BlockSpec((1, 128), lambda i: (0, 0)),
            pl.BlockSpec((16, 128), lambda i: (0, 0)),
            pl.BlockSpec((1, 128), lambda i: (0, 0)),
            pl.BlockSpec((1, 128), lambda i: (0, 0)),
            pl.BlockSpec((1, 128), lambda i: (0, 0)),
        ],
        out_specs=[pl.BlockSpec((BN, 128), lambda i: (i, 0))] * 2,
        out_shape=[jax.ShapeDtypeStruct((n, 128), F32)] * 2,
    )(feat, pos, lnw, lnb.reshape(1, -1), lpw, lpb.reshape(1, -1),
      g.reshape(1, -1), b.reshape(1, -1))


def _final(hc, w1, b1, sm, tm, w2, b2):
    n = hc.shape[0]

    def body(x_ref, w1_ref, b1_ref, sm_ref, tm_ref, w2_ref, b2_ref, o_ref):
        s = jnp.sum(x_ref[...], axis=0, keepdims=True)
        y = (
            jnp.dot(s, w1_ref[...], preferred_element_type=F32) + b1_ref[...]
        ) * sm_ref[...] + tm_ref[...]
        y = jnp.where(y > 0.0, y, jnp.exp(y) - 1.0)
        o_ref[...] = (
            jnp.dot(y, w2_ref[...], preferred_element_type=F32) + b2_ref[...]
        )

    return pl.pallas_call(
        body,
        grid=(1,),
        in_specs=[
            pl.BlockSpec((n, 128), lambda i: (0, 0)),
            pl.BlockSpec((128, 128), lambda i: (0, 0)),
            pl.BlockSpec((1, 128), lambda i: (0, 0)),
            pl.BlockSpec((1, 128), lambda i: (0, 0)),
            pl.BlockSpec((1, 128), lambda i: (0, 0)),
            pl.BlockSpec((128, 1), lambda i: (0, 0)),
            pl.BlockSpec((1, 1), lambda i: (0, 0)),
        ],
        out_specs=pl.BlockSpec((1, 1), lambda i: (0, 0)),
        out_shape=jax.ShapeDtypeStruct((1, 1), F32),
    )(hc, w1, b1.reshape(1, -1), sm.reshape(1, -1), tm.reshape(1, -1),
      w2, b2.reshape(1, -1))


def _sc_gather(dst, src, t_dst, tables_src):
    e = dst.shape[0]
    ew = e // NW
    ch = _chunk(ew)
    nit = ew // ch
    ksrc = len(tables_src)
    k = 1 + ksrc
    mesh = plsc.VectorSubcoreMesh(core_axis_name="c", subcore_axis_name="s")

    def body(*refs):
        dst_h, src_h = refs[0], refs[1]
        tbls = refs[2:2 + k]
        outs = refs[2 + k:2 + 2 * k]
        scr = refs[2 + 2 * k:]
        idxd, idxs = scr[0], scr[1]
        bufs = scr[2:2 + k]
        sems = scr[2 + k:]
        wid = lax.axis_index("s") * NC + lax.axis_index("c")
        base = wid * ew

        def step(i, carry):
            off = base + i * ch
            pltpu.sync_copy(dst_h.at[pl.ds(off, ch)], idxd)
            pltpu.sync_copy(src_h.at[pl.ds(off, ch)], idxs)
            cps = []
            for j in range(k):
                idx = idxd if j == 0 else idxs
                cps.append(pltpu.async_copy(tbls[j].at[idx], bufs[j], sems[j]))
            for j in range(k):
                cps[j].wait()
                pltpu.sync_copy(bufs[j], outs[j].at[pl.ds(off, ch)])
            return carry

        lax.fori_loop(0, nit, step, 0)

    fn = pl.kernel(
        body,
        out_type=tuple(jax.ShapeDtypeStruct((e, 128), F32) for _ in range(k)),
        mesh=mesh,
        scratch_types=(
            [pltpu.VMEM((ch,), jnp.int32)] * 2
            + [pltpu.VMEM((ch, 128), F32) for _ in range(k)]
            + [pltpu.SemaphoreType.DMA for _ in range(k)]
        ),
    )
    return fn(dst, src, t_dst, *tables_src)


def _chunk_cap(m, cap):
    best = 8
    for c in range(8, cap + 1, 8):
        if m % c == 0:
            best = c
    return best


def _sc_k1(dst, src, b1h, b2h, b3, se, te, n, zrows, want_enx):
    e = dst.shape[0]
    n_pad = 128 * ((n + 127) // 128)
    eh = e // NC
    et = eh // NS
    ch = _chunk_cap(et, 40)
    nit = et // ch
    mesh = plsc.VectorSubcoreMesh(core_axis_name="c", subcore_axis_name="s")

    def body(*refs):
        (dst_h, src_h, b1_h, b2_h, b3_h, se_h, te_h, z_h) = refs[:8]
        outs = refs[8:8 + (3 if want_enx else 2)]
        scr = refs[8 + (3 if want_enx else 2):]
        ssum_h, sig_h = outs[0], outs[1]
        enx_h = outs[2] if want_enx else None
        idxd, idxs = scr[0], scr[1]
        bufd, bufs, bufb, sigb = scr[2], scr[3], scr[4], scr[5]
        i = 6
        enxb = scr[i] if want_enx else None
        i += 1 if want_enx else 0
        sev, tev = scr[i], scr[i + 1]
        acc = scr[i + 2]
        semd, sems = scr[i + 3], scr[i + 4]
        cid = lax.axis_index("c")
        sid = lax.axis_index("s")

        rpt = n_pad // NS
        r0 = sid * rpt
        off = 0
        while off < rpt:
            sz = min(128, rpt - off)
            pltpu.sync_copy(z_h.at[pl.ds(0, sz)], acc.at[pl.ds(r0 + off, sz)])
            off += sz
        pltpu.sync_copy(se_h, sev)
        pltpu.sync_copy(te_h, tev)
        plsc.subcore_barrier()

        sejs = [sev[pl.ds(j * 16, 16)] for j in range(8)]
        tejs = [tev[pl.ds(j * 16, 16)] for j in range(8)]

        tbase = cid * eh + sid * et

        def step(it, carry):
            o = tbase + it * ch
            pltpu.sync_copy(dst_h.at[pl.ds(o, ch)], idxd)
            pltpu.sync_copy(src_h.at[pl.ds(o, ch)], idxs)
            cpd = pltpu.async_copy(b1_h.at[idxd], bufd, semd)
            cps = pltpu.async_copy(b2_h.at[idxs], bufs, sems)
            pltpu.sync_copy(b3_h.at[pl.ds(o, ch)], bufb)
            cpd.wait()
            cps.wait()
            for r in range(ch):
                for j in range(8):
                    s16 = pl.ds(j * 16, 16)
                    hat = bufd[r, s16] + bufs[r, s16] + bufb[r, s16]
                    sg = 1.0 / (1.0 + jnp.exp(-hat))
                    sigb[r, s16] = sg
                    if want_enx:
                        enxb[r, s16] = jnp.maximum(
                            hat * sejs[j] + tejs[j], 0.0)
            pltpu.sync_copy(sigb, sig_h.at[pl.ds(o, ch)])
            if want_enx:
                pltpu.sync_copy(enxb, enx_h.at[pl.ds(o, ch)])
            pltpu.sync_copy(sigb, acc.at[idxd], add=True)
            return carry

        lax.fori_loop(0, nit, step, 0)
        plsc.subcore_barrier()

        o = 0
        while o < rpt:
            sz = min(128, rpt - o)
            pltpu.sync_copy(
                acc.at[pl.ds(r0 + o, sz)],
                ssum_h.at[cid, pl.ds(r0 + o, sz)],
            )
            o += sz

    out_type = [
        jax.ShapeDtypeStruct((NC, n_pad, 128), F32),
        jax.ShapeDtypeStruct((e, 128), F32),
    ]
    if want_enx:
        out_type.append(jax.ShapeDtypeStruct((e, 128), F32))
    scratch = [
        pltpu.VMEM((ch,), jnp.int32),
        pltpu.VMEM((ch,), jnp.int32),
        pltpu.VMEM((ch, 128), F32),
        pltpu.VMEM((ch, 128), F32),
        pltpu.VMEM((ch, 128), F32),
        pltpu.VMEM((ch, 128), F32),
    ]
    if want_enx:
        scratch.append(pltpu.VMEM((ch, 128), F32))
    scratch += [
        pltpu.VMEM((128,), F32),
        pltpu.VMEM((128,), F32),
        pltpu.VMEM_SHARED((n_pad, 128), F32),
        pltpu.SemaphoreType.DMA,
        pltpu.SemaphoreType.DMA,
    ]
    fn = pl.kernel(
        body,
        out_type=tuple(out_type),
        mesh=mesh,
        scratch_types=tuple(scratch),
    )
    res = fn(dst, src, b1h, b2h, b3, se, te, zrows)
    ssum = res[0][:, :n] if n_pad != n else res[0]
    return (ssum,) + tuple(res[1:])


def _sc_k2(dst, src, table, sig, n, zrows):
    e = dst.shape[0]
    n_pad = 128 * ((n + 127) // 128)
    eh = e // NC
    et = eh // NS
    ch = _chunk_cap(et, 80)
    nit = et // ch
    mesh = plsc.VectorSubcoreMesh(core_axis_name="c", subcore_axis_name="s")

    def body(dst_h, src_h, tab_h, sig_h, z_h, out_h,
             idxd, idxs, bufv, bufsg, acc, semv):
        cid = lax.axis_index("c")
        sid = lax.axis_index("s")

        rpt = n_pad // NS
        r0 = sid * rpt
        off = 0
        while off < rpt:
            sz = min(128, rpt - off)
            pltpu.sync_copy(z_h.at[pl.ds(0, sz)], acc.at[pl.ds(r0 + off, sz)])
            off += sz
        plsc.subcore_barrier()

        tbase = cid * eh + sid * et

        def step(it, carry):
            o = tbase + it * ch
            pltpu.sync_copy(dst_h.at[pl.ds(o, ch)], idxd)
            pltpu.sync_copy(src_h.at[pl.ds(o, ch)], idxs)
            cpv = pltpu.async_copy(tab_h.at[idxs], bufv, semv)
            pltpu.sync_copy(sig_h.at[pl.ds(o, ch)], bufsg)
            cpv.wait()
            for r in range(ch):
                for j in range(8):
                    s16 = pl.ds(j * 16, 16)
                    bufsg[r, s16] = bufsg[r, s16] * bufv[r, s16]
            pltpu.sync_copy(bufsg, acc.at[idxd], add=True)
            return carry

        lax.fori_loop(0, nit, step, 0)
        plsc.subcore_barrier()

        o = 0
        while o < rpt:
            sz = min(128, rpt - o)
            pltpu.sync_copy(
                acc.at[pl.ds(r0 + o, sz)],
                out_h.at[cid, pl.ds(r0 + o, sz)],
            )
            o += sz

    fn = pl.kernel(
        body,
        out_type=jax.ShapeDtypeStruct((NC, n_pad, 128), F32),
        mesh=mesh,
        scratch_types=(
            pltpu.VMEM((ch,), jnp.int32),
            pltpu.VMEM((ch,), jnp.int32),
            pltpu.VMEM((ch, 128), F32),
            pltpu.VMEM((ch, 128), F32),
            pltpu.VMEM_SHARED((n_pad, 128), F32),
            pltpu.SemaphoreType.DMA,
        ),
    )
    out = fn(dst, src, table, sig, zrows)
    return out[:, :n] if n_pad != n else out


def _sc_scatter_es(vals, dst, n, zrows):
    e = vals.shape[0]
    n_pad = 128 * ((n + 127) // 128)
    eh = e // NC
    et = eh // NS
    ch = _chunk(et)
    nit = et // ch
    mesh = plsc.VectorSubcoreMesh(core_axis_name="c", subcore_axis_name="s")

    def body(vals_h, dst_h, z_h, out_h, idxv, buf, acc):
        cid = lax.axis_index("c")
        sid = lax.axis_index("s")

        rpt = n_pad // NS
        r0 = sid * rpt
        off = 0
        while off < rpt:
            sz = min(128, rpt - off)
            pltpu.sync_copy(z_h.at[pl.ds(0, sz)], acc.at[pl.ds(r0 + off, sz)])
            off += sz
        plsc.subcore_barrier()

        tbase = cid * eh + sid * et

        def step(i, carry):
            o = tbase + i * ch
            pltpu.sync_copy(dst_h.at[pl.ds(o, ch)], idxv)
            pltpu.sync_copy(vals_h.at[pl.ds(o, ch)], buf)
            pltpu.sync_copy(buf, acc.at[idxv], add=True)
            return carry

        lax.fori_loop(0, nit, step, 0)
        plsc.subcore_barrier()

        o = 0
        while o < rpt:
            sz = min(128, rpt - o)
            pltpu.sync_copy(
                acc.at[pl.ds(r0 + o, sz)],
                out_h.at[cid, pl.ds(r0 + o, sz)],
            )
            o += sz

    fn = pl.kernel(
        body,
        out_type=jax.ShapeDtypeStruct((NC, n_pad, 128), F32),
        mesh=mesh,
        scratch_types=(
            pltpu.VMEM((ch,), jnp.int32),
            pltpu.VMEM((ch, 128), F32),
            pltpu.VMEM_SHARED((n_pad, 128), F32),
        ),
    )
    out = fn(vals, dst, zrows)
    return out[:, :n] if n_pad != n else out


def _sc_scatter(vals, dst, n, zrows):
    e = vals.shape[0]
    n_pad = 2048 * ((n + 2047) // 2048)
    nrm = n_pad // 2
    rows = nrm + 128
    junk = nrm
    et = e // NS
    ch = _chunk(et)
    nit = et // ch
    mesh = plsc.VectorSubcoreMesh(core_axis_name="c", subcore_axis_name="s")

    def body(vals_h, dst_h, z_h, out_h, idxv, locv, buf, acc):
        cid = lax.axis_index("c")
        sid = lax.axis_index("s")
        rbase = (cid * nrm).astype(jnp.int32)

        rpt_i = rows // NS
        r0 = sid * rpt_i
        off = 0
        while off < rpt_i:
            sz = min(128, rpt_i - off)
            pltpu.sync_copy(z_h.at[pl.ds(0, sz)], acc.at[pl.ds(r0 + off, sz)])
            off += sz
        plsc.subcore_barrier()

        tbase = sid * et

        def step(i, carry):
            o = tbase + i * ch
            pltpu.sync_copy(dst_h.at[pl.ds(o, ch)], idxv)
            pltpu.sync_copy(vals_h.at[pl.ds(o, ch)], buf)
            for j in range(ch // 16):
                t = idxv[pl.ds(j * 16, 16)] - rbase
                ok = (t >= 0) & (t < nrm)
                locv[pl.ds(j * 16, 16)] = jnp.where(ok, t, junk)
            pltpu.sync_copy(buf, acc.at[locv], add=True)
            return carry

        lax.fori_loop(0, nit, step, 0)
        plsc.subcore_barrier()

        rpt = nrm // NS
        rr0 = sid * rpt
        o = 0
        while o < rpt:
            sz = min(128, rpt - o)
            pltpu.sync_copy(
                acc.at[pl.ds(rr0 + o, sz)],
                out_h.at[pl.ds(rbase + rr0 + o, sz)],
            )
            o += sz

    fn = pl.kernel(
        body,
        out_type=jax.ShapeDtypeStruct((n_pad, 128), F32),
        mesh=mesh,
        scratch_types=(
            pltpu.VMEM((ch,), jnp.int32),
            pltpu.VMEM((ch,), jnp.int32),
            pltpu.VMEM((ch, 128), F32),
            pltpu.VMEM_SHARED((rows, 128), F32),
        ),
    )
    out = fn(vals, dst, zrows)
    return out[:n] if n_pad != n else out


def _fold_bn(g, b, m, v):
    s = g / jnp.sqrt(v + 1e-5)
    return s, b - m * s


def _gated(lp, src, dst, h, p, b3, n, zrows, want_p, want_enx, res=None):
    if want_p:
        wh = jnp.concatenate(
            [lp["B1_W"], lp["B2_W"], lp["A1_W"][:128], lp["A2_W"][:128],
             jnp.zeros((128, 256), F32)], axis=1)
        wp = jnp.concatenate(
            [jnp.zeros((128, 256), F32), lp["A1_W"][128:], lp["A2_W"][128:],
             lp["C1_W"], lp["C2_W"]], axis=1)
        bb = jnp.concatenate(
            [lp["B1_b"], lp["B2_b"], lp["A1_b"], lp["A2_b"],
             lp["C1_b"], lp["C2_b"]])
        b1h, b2h, a1, v, c1, cp = _node_dense(h, p, wh, wp, bb)
    else:
        wh = jnp.concatenate(
            [lp["B1_W"], lp["B2_W"], lp["A1_W"][:128], lp["A2_W"][:128]],
            axis=1)
        wp = jnp.concatenate(
            [jnp.zeros((128, 256), F32), lp["A1_W"][128:], lp["A2_W"][128:]],
            axis=1)
        bb = jnp.concatenate(
            [lp["B1_b"], lp["B2_b"], lp["A1_b"], lp["A2_b"]])
        b1h, b2h, a1, v = _node_dense(h, p, wh, wp, bb)
        c1 = cp = None

    se, te = _fold_bn(lp["bne_g"], lp["bne_b"], lp["bne_m"], lp["bne_v"])
    sh, th = _fold_bn(lp["bnh_g"], lp["bnh_b"], lp["bnh_m"], lp["bnh_v"])

    if want_p:
        bd, bs = _sc_gather(dst, src, b1h, [b2h])
        ef = _edge_fuse(bd, bs, b3, None, None, se, te, want_enx)
        sig = ef[0]
        enx = ef[1] if want_enx else None
        ssum = _sc_scatter_es(sig, dst, n, zrows)
        sev = _sc_k2(dst, src, v, sig, n, zrows)
        sep = _sc_k2(dst, src, cp, sig, n, zrows)
        nu = _node_update(ssum, sev, sep, a1, c1, sh, th, res, True)
        return nu[0], nu[1], enx

    tables = [b2h, v]
    gathered = _sc_gather(dst, src, b1h, tables)
    bd, bs, vg = gathered[0], gathered[1], gathered[2]

    ef = _edge_fuse(bd, bs, b3, vg, None, se, te, want_enx)
    sig, ev = ef[0], ef[1]
    enx = ef[2] if want_enx else None

    stacked = n <= 16384
    scat = _sc_scatter_es if stacked else _sc_scatter
    ssum = scat(sig, dst, n, zrows)
    sev = scat(ev, dst, n, zrows)

    nu = _node_update(ssum, sev, None, a1, c1, sh, th, res, stacked)
    return nu[0], None, enx


def kernel(gp_token_res, gp_token_atom, gp_pos_enc, gp_dist, gp_edge_index,
           gl_feat, gl_pos_enc, gl_edge_feat, gl_edge_index,
           gc_dist, gc_edge_index, params):
    pr = params
    n_p = gp_token_res.shape[0]
    n_l = gl_feat.shape[0]
    n_c = n_p + n_l

    zrows = jnp.zeros((128, 128), F32)

    tok = jnp.stack(
        [gp_token_res.astype(jnp.int32), gp_token_atom.astype(jnp.int32)],
        axis=1)
    res_pad = jnp.pad(pr["res_emb"], ((0, 2), (0, 0)))
    atom_pad = jnp.pad(pr["atom_emb"], ((0, 1), (0, 0)))
    hp, pp = _embed_p(tok, gp_pos_enc, res_pad, atom_pad,
                      pr["pp_W"], pr["pp_b"], pr["pnorm_g"], pr["pnorm_b"])
    hl, pl_ = _embed_l(gl_feat, gl_pos_enc, pr["ln_W"], pr["ln_b"],
                       pr["lp_W"], pr["lp_b"], pr["lnorm_g"], pr["lnorm_b"])
    hp_raw, hl_raw = hp, hl
    res_c = jnp.concatenate([hp_raw, hl_raw], axis=0)

    ps, pd = gp_edge_index[0], gp_edge_index[1]
    ls, ld = gl_edge_index[0], gl_edge_index[1]
    cs, cd = gc_edge_index[0], gc_edge_index[1]

    xp = jnp.pad(gp_dist, ((0, 0), (0, 1)))
    xc = jnp.pad(gc_dist, ((0, 0), (0, 1)))
    pe_w = jnp.pad(pr["pe_W"], ((0, 1), (0, 0)))
    ce_w = jnp.pad(pr["ce_W"], ((0, 1), (0, 0)))

    def b3_first(x, ew, eb, blk):
        return _mm(_mm(x, ew, eb), blk["B3_W"], blk["B3_b"])

    enx_p = enx_l = enx_c = None
    hc = None
    for i in range(3):
        bp, bl, bc = pr["pblock"][i], pr["lblock"][i], pr["cblock"][i]
        last = i == 2

        if i == 0:
            b3p = b3_first(xp, pe_w, pr["pe_b"], bp)
            b3l = b3_first(gl_edge_feat, pr["le_W"], pr["le_b"], bl)
        else:
            b3p = _mm(enx_p, bp["B3_W"], bp["B3_b"])
            b3l = _mm(enx_l, bl["B3_W"], bl["B3_b"])
        hp, pp, enx_p = _gated(bp, ps, pd, hp, pp, b3p, n_p, zrows,
                               want_p=True, want_enx=not last)
        hl, pl_, enx_l = _gated(bl, ls, ld, hl, pl_, b3l, n_l, zrows,
                                want_p=True, want_enx=not last)

        hcat = jnp.concatenate([hp, hl], axis=0)
        pcat = jnp.concatenate([pp, pl_], axis=0)
        if i == 0:
            b3c = b3_first(xc, ce_w, pr["ce_b"], bc)
        else:
            b3c = _mm(enx_c, bc["B3_W"], bc["B3_b"])
        hc, _, enx_c = _gated(bc, cs, cd, hcat, pcat, b3c, n_c, zrows,
                              want_p=False, want_enx=not last,
                              res=None if last else res_c)
        if not last:
            hp = hc[:n_p]
            hl = hc[n_p:]

    sm, tm = _fold_bn(pr["mbn_g"], pr["mbn_b"], pr["mbn_m"], pr["mbn_v"])
    return _final(hc, pr["mlp1_W"], pr["mlp1_b"], sm, tm,
                  pr["mlp2_W"], pr["mlp2_b"])


# speedup vs baseline: 1.4649x; 1.1048x over previous
import functools

import jax
import jax.numpy as jnp
from jax import lax
from jax.experimental import pallas as pl
from jax.experimental.pallas import tpu as pltpu
from jax.experimental.pallas import tpu_sc as plsc

F32 = jnp.float32
NC, NS, NL = 2, 16, 16
NW = NC * NS
BN = 2000


def _chunk(m):
    best = 8
    for c in range(8, 129, 8):
        if m % c == 0:
            best = c
    return best


def _mm(x, w, b):
    n, k = x.shape
    m = w.shape[1]

    def body(x_ref, w_ref, b_ref, o_ref):
        o_ref[...] = (
            jnp.dot(x_ref[...], w_ref[...], preferred_element_type=F32)
            + b_ref[...]
        )

    return pl.pallas_call(
        body,
        grid=(n // BN,),
        in_specs=[
            pl.BlockSpec((BN, k), lambda i: (i, 0)),
            pl.BlockSpec((k, m), lambda i: (0, 0)),
            pl.BlockSpec((1, m), lambda i: (0, 0)),
        ],
        out_specs=pl.BlockSpec((BN, m), lambda i: (i, 0)),
        out_shape=jax.ShapeDtypeStruct((n, m), F32),
    )(x, w, b.reshape(1, -1))


def _node_dense(h, p, wh, wp, b):
    n = h.shape[0]
    m = wh.shape[1]
    nout = m // 128

    def body(h_ref, p_ref, wh_ref, wp_ref, b_ref, *outs):
        y = (
            jnp.dot(h_ref[...], wh_ref[...], preferred_element_type=F32)
            + jnp.dot(p_ref[...], wp_ref[...], preferred_element_type=F32)
            + b_ref[...]
        )
        for j, o_ref in enumerate(outs):
            o_ref[...] = y[:, j * 128:(j + 1) * 128]

    return pl.pallas_call(
        body,
        grid=(n // BN,),
        in_specs=[
            pl.BlockSpec((BN, 128), lambda i: (i, 0)),
            pl.BlockSpec((BN, 128), lambda i: (i, 0)),
            pl.BlockSpec((128, m), lambda i: (0, 0)),
            pl.BlockSpec((128, m), lambda i: (0, 0)),
            pl.BlockSpec((1, m), lambda i: (0, 0)),
        ],
        out_specs=[pl.BlockSpec((BN, 128), lambda i: (i, 0))] * nout,
        out_shape=[jax.ShapeDtypeStruct((n, 128), F32)] * nout,
    )(h, p, wh, wp, b.reshape(1, -1))


def _edge_fuse(bd, bs, b3, vg, cg, se, te, want_enx):
    e = bd.shape[0]
    want_p = cg is not None
    nin = 5 if want_p else 4
    nout = 2 + (1 if want_p else 0) + (1 if want_enx else 0)

    def body(*refs):
        ins = refs[:nin]
        se_ref, te_ref = refs[nin], refs[nin + 1]
        outs = refs[nin + 2:]
        hat = ins[0][...] + ins[1][...] + ins[2][...]
        sig = jax.nn.sigmoid(hat)
        res = [sig, sig * ins[3][...]]
        if want_p:
            res.append(sig * ins[4][...])
        if want_enx:
            res.append(jnp.maximum(hat * se_ref[...] + te_ref[...], 0.0))
        for o_ref, val in zip(outs, res):
            o_ref[...] = val

    args = [bd, bs, b3, vg] + ([cg] if want_p else [])
    return pl.pallas_call(
        body,
        grid=(e // BN,),
        in_specs=[pl.BlockSpec((BN, 128), lambda i: (i, 0))] * nin
        + [pl.BlockSpec((1, 128), lambda i: (0, 0))] * 2,
        out_specs=[pl.BlockSpec((BN, 128), lambda i: (i, 0))] * nout,
        out_shape=[jax.ShapeDtypeStruct((e, 128), F32)] * nout,
    )(*args, se.reshape(1, -1), te.reshape(1, -1))


def _node_update(ssum, sev, sep, a1, c1, sh, th, res, stacked):
    want_p = sep is not None
    have_res = res is not None
    nseg = 3 if want_p else 2
    n = ssum.shape[1] if stacked else ssum.shape[0]
    nin = nseg + (2 if want_p else 1) + (1 if have_res else 0)

    def body(*refs):
        i = 0
        segs = []
        for _ in range(nseg):
            r_ = refs[i]; i += 1
            segs.append(r_[0] + r_[1] if stacked else r_[...])
        a1_ref = refs[i]; i += 1
        if want_p:
            c1_ref = refs[i]; i += 1
        if have_res:
            res_ref = refs[i]; i += 1
        sh_ref = refs[i]; i += 1
        th_ref = refs[i]; i += 1
        outs = refs[i:]
        r = 1.0 / (segs[0] + 1e-6)
        h = jnp.maximum(
            (a1_ref[...] + r * segs[1]) * sh_ref[...] + th_ref[...], 0.0
        )
        if have_res:
            h = h + res_ref[...]
        outs[0][...] = h
        if want_p:
            outs[1][...] = jnp.tanh(c1_ref[...] + r * segs[2])

    args = [ssum, sev] + ([sep] if want_p else []) + [a1] \
        + ([c1] if want_p else [])
    if have_res:
        args += [res]
    nout = 2 if want_p else 1
    seg_spec = (
        pl.BlockSpec((NC, BN, 128), lambda i: (0, i, 0))
        if stacked else pl.BlockSpec((BN, 128), lambda i: (i, 0))
    )
    return pl.pallas_call(
        body,
        grid=(n // BN,),
        in_specs=[seg_spec] * nseg
        + [pl.BlockSpec((BN, 128), lambda i: (i, 0))] * (nin - nseg)
        + [pl.BlockSpec((1, 128), lambda i: (0, 0))] * 2,
        out_specs=[pl.BlockSpec((BN, 128), lambda i: (i, 0))] * nout,
        out_shape=[jax.ShapeDtypeStruct((n, 128), F32)] * nout,
    )(*args, sh.reshape(1, -1), th.reshape(1, -1))


def _embed_p(tok, pos, res_emb, atom_emb, ppw, ppb, g, b):
    n = tok.shape[0]
    kr = res_emb.shape[0]
    ka = atom_emb.shape[0]

    def body(tok_ref, pos_ref, re_ref, ae_ref, ppw_ref, ppb_ref, g_ref,
             b_ref, h_ref, pp_ref):
        tr = tok_ref[:, 0:1]
        ta = tok_ref[:, 1:2]
        ohr = (tr == lax.broadcasted_iota(jnp.int32, (1, kr), 1)).astype(F32)
        oha = (ta == lax.broadcasted_iota(jnp.int32, (1, ka), 1)).astype(F32)
        hr = jnp.dot(ohr, re_ref[...], preferred_element_type=F32,
                     precision=lax.Precision.HIGHEST)
        ha = jnp.dot(oha, ae_ref[...], preferred_element_type=F32,
                     precision=lax.Precision.HIGHEST)
        x = jnp.concatenate([hr, ha], axis=1)
        mu = jnp.mean(x, axis=-1, keepdims=True)
        var = jnp.mean((x - mu) ** 2, axis=-1, keepdims=True)
        h_ref[...] = (x - mu) / jnp.sqrt(var + 1e-5) * g_ref[...] + b_ref[...]
        pp_ref[...] = (
            jnp.dot(pos_ref[...], ppw_ref[...], preferred_element_type=F32)
            + ppb_ref[...]
        )

    return pl.pallas_call(
        body,
        grid=(n // BN,),
        in_specs=[
            pl.BlockSpec((BN, 2), lambda i: (i, 0)),
            pl.BlockSpec((BN, 16), lambda i: (i, 0)),
            pl.BlockSpec((kr, 64), lambda i: (0, 0)),
            pl.BlockSpec((ka, 64), lambda i: (0, 0)),
            pl.BlockSpec((16, 128), lambda i: (0, 0)),
            pl.BlockSpec((1, 128), lambda i: (0, 0)),
            pl.BlockSpec((1, 128), lambda i: (0, 0)),
            pl.BlockSpec((1, 128), lambda i: (0, 0)),
        ],
        out_specs=[pl.BlockSpec((BN, 128), lambda i: (i, 0))] * 2,
        out_shape=[jax.ShapeDtypeStruct((n, 128), F32)] * 2,
    )(tok, pos, res_emb, atom_emb, ppw, ppb.reshape(1, -1),
      g.reshape(1, -1), b.reshape(1, -1))


def _embed_l(feat, pos, lnw, lnb, lpw, lpb, g, b):
    n = feat.shape[0]

    def body(f_ref, pos_ref, lnw_ref, lnb_ref, lpw_ref, lpb_ref, g_ref,
             b_ref, h_ref, pp_ref):
        x = (
            jnp.dot(f_ref[...], lnw_ref[...], preferred_element_type=F32)
            + lnb_ref[...]
        )
        mu = jnp.mean(x, axis=-1, keepdims=True)
        var = jnp.mean((x - mu) ** 2, axis=-1, keepdims=True)
        h_ref[...] = (x - mu) / jnp.sqrt(var + 1e-5) * g_ref[...] + b_ref[...]
        pp_ref[...] = (
            jnp.dot(pos_ref[...], lpw_ref[...], preferred_element_type=F32)
            + lpb_ref[...]
        )

    return pl.pallas_call(
        body,
        grid=(n // BN,),
        in_specs=[
            pl.BlockSpec((BN, 128), lambda i: (i, 0)),
            pl.BlockSpec((BN, 16), lambda i: (i, 0)),
            pl.BlockSpec((128, 128), lambda i: (0, 0)),
            pl.BlockSpec((1, 128), lambda i: (0, 0)),
            pl.BlockSpec((16, 128), lambda i: (0, 0)),
            pl.BlockSpec((1, 128), lambda i: (0, 0)),
            pl.BlockSpec((1, 128), lambda i: (0, 0)),
            pl.BlockSpec((1, 128), lambda i: (0, 0)),
        ],
        out_specs=[pl.BlockSpec((BN, 128), lambda i: (i, 0))] * 2,
        out_shape=[jax.ShapeDtypeStruct((n, 128), F32)] * 2,
    )(feat, pos, lnw, lnb.reshape(1, -1), lpw, lpb.reshape(1, -1),
      g.reshape(1, -1), b.reshape(1, -1))


def _final(hc, w1, b1, sm, tm, w2, b2):
    n = hc.shape[0]

    def body(x_ref, w1_ref, b1_ref, sm_ref, tm_ref, w2_ref, b2_ref, o_ref):
        s = jnp.sum(x_ref[...], axis=0, keepdims=True)
        y = (
            jnp.dot(s, w1_ref[...], preferred_element_type=F32) + b1_ref[...]
        ) * sm_ref[...] + tm_ref[...]
        y = jnp.where(y > 0.0, y, jnp.exp(y) - 1.0)
        o_ref[...] = (
            jnp.dot(y, w2_ref[...], preferred_element_type=F32) + b2_ref[...]
        )

    return pl.pallas_call(
        body,
        grid=(1,),
        in_specs=[
            pl.BlockSpec((n, 128), lambda i: (0, 0)),
            pl.BlockSpec((128, 128), lambda i: (0, 0)),
            pl.BlockSpec((1, 128), lambda i: (0, 0)),
            pl.BlockSpec((1, 128), lambda i: (0, 0)),
            pl.BlockSpec((1, 128), lambda i: (0, 0)),
            pl.BlockSpec((128, 1), lambda i: (0, 0)),
            pl.BlockSpec((1, 1), lambda i: (0, 0)),
        ],
        out_specs=pl.BlockSpec((1, 1), lambda i: (0, 0)),
        out_shape=jax.ShapeDtypeStruct((1, 1), F32),
    )(hc, w1, b1.reshape(1, -1), sm.reshape(1, -1), tm.reshape(1, -1),
      w2, b2.reshape(1, -1))


def _sc_gather(dst, src, t_dst, tables_src):
    e = dst.shape[0]
    ew = e // NW
    ch = _chunk(ew)
    nit = ew // ch
    ksrc = len(tables_src)
    k = 1 + ksrc
    mesh = plsc.VectorSubcoreMesh(core_axis_name="c", subcore_axis_name="s")

    def body(*refs):
        dst_h, src_h = refs[0], refs[1]
        tbls = refs[2:2 + k]
        outs = refs[2 + k:2 + 2 * k]
        scr = refs[2 + 2 * k:]
        idxd, idxs = scr[0], scr[1]
        bufs = scr[2:2 + k]
        sems = scr[2 + k:]
        wid = lax.axis_index("s") * NC + lax.axis_index("c")
        base = wid * ew

        def step(i, carry):
            off = base + i * ch
            pltpu.sync_copy(dst_h.at[pl.ds(off, ch)], idxd)
            pltpu.sync_copy(src_h.at[pl.ds(off, ch)], idxs)
            cps = []
            for j in range(k):
                idx = idxd if j == 0 else idxs
                cps.append(pltpu.async_copy(tbls[j].at[idx], bufs[j], sems[j]))
            for j in range(k):
                cps[j].wait()
                pltpu.sync_copy(bufs[j], outs[j].at[pl.ds(off, ch)])
            return carry

        lax.fori_loop(0, nit, step, 0)

    fn = pl.kernel(
        body,
        out_type=tuple(jax.ShapeDtypeStruct((e, 128), F32) for _ in range(k)),
        mesh=mesh,
        scratch_types=(
            [pltpu.VMEM((ch,), jnp.int32)] * 2
            + [pltpu.VMEM((ch, 128), F32) for _ in range(k)]
            + [pltpu.SemaphoreType.DMA for _ in range(k)]
        ),
    )
    return fn(dst, src, t_dst, *tables_src)


def _sc_scatter_es_multi(vals_list, dst, n, zrows):
    t_n = len(vals_list)
    e = vals_list[0].shape[0]
    n_pad = 128 * ((n + 127) // 128)
    eh = e // NC
    et = eh // NS
    ch = _chunk(et)
    nit = et // ch
    mesh = plsc.VectorSubcoreMesh(core_axis_name="c", subcore_axis_name="s")

    def body(*refs):
        vals_h = refs[:t_n]
        dst_h, z_h = refs[t_n], refs[t_n + 1]
        outs_h = refs[t_n + 2:2 * t_n + 2]
        idxv, buf = refs[2 * t_n + 2], refs[2 * t_n + 3]
        acc = refs[2 * t_n + 4]
        cid = lax.axis_index("c")
        sid = lax.axis_index("s")
        rpt = n_pad // NS
        r0 = sid * rpt
        tbase = cid * eh + sid * et

        for t in range(t_n):
            off = 0
            while off < rpt:
                sz = min(128, rpt - off)
                pltpu.sync_copy(z_h.at[pl.ds(0, sz)],
                                acc.at[pl.ds(r0 + off, sz)])
                off += sz
            plsc.subcore_barrier()

            def step(i, carry):
                o = tbase + i * ch
                pltpu.sync_copy(dst_h.at[pl.ds(o, ch)], idxv)
                pltpu.sync_copy(vals_h[t].at[pl.ds(o, ch)], buf)
                pltpu.sync_copy(buf, acc.at[idxv], add=True)
                return carry

            lax.fori_loop(0, nit, step, 0)
            plsc.subcore_barrier()

            o = 0
            while o < rpt:
                sz = min(128, rpt - o)
                pltpu.sync_copy(
                    acc.at[pl.ds(r0 + o, sz)],
                    outs_h[t].at[cid, pl.ds(r0 + o, sz)],
                )
                o += sz

    fn = pl.kernel(
        body,
        out_type=tuple(
            jax.ShapeDtypeStruct((NC, n_pad, 128), F32) for _ in range(t_n)),
        mesh=mesh,
        scratch_types=(
            pltpu.VMEM((ch,), jnp.int32),
            pltpu.VMEM((ch, 128), F32),
            pltpu.VMEM_SHARED((n_pad, 128), F32),
        ),
    )
    outs = fn(*vals_list, dst, zrows)
    if n_pad != n:
        outs = tuple(o[:, :n] for o in outs)
    return outs


def _sc_scatter_multi(vals_list, dst, n, zrows):
    t_n = len(vals_list)
    e = vals_list[0].shape[0]
    n_pad = 2048 * ((n + 2047) // 2048)
    nrm = n_pad // 2
    rows = nrm + 128
    junk = nrm
    et = e // NS
    ch = _chunk(et)
    nit = et // ch
    mesh = plsc.VectorSubcoreMesh(core_axis_name="c", subcore_axis_name="s")

    def body(*refs):
        vals_h = refs[:t_n]
        dst_h, z_h = refs[t_n], refs[t_n + 1]
        outs_h = refs[t_n + 2:2 * t_n + 2]
        idxv, locv = refs[2 * t_n + 2], refs[2 * t_n + 3]
        buf, acc = refs[2 * t_n + 4], refs[2 * t_n + 5]
        cid = lax.axis_index("c")
        sid = lax.axis_index("s")
        rbase = (cid * nrm).astype(jnp.int32)
        rpt_i = rows // NS
        r0 = sid * rpt_i
        tbase = sid * et
        rpt = nrm // NS
        rr0 = sid * rpt

        for t in range(t_n):
            off = 0
            while off < rpt_i:
                sz = min(128, rpt_i - off)
                pltpu.sync_copy(z_h.at[pl.ds(0, sz)],
                                acc.at[pl.ds(r0 + off, sz)])
                off += sz
            plsc.subcore_barrier()

            def step(i, carry):
                o = tbase + i * ch
                pltpu.sync_copy(dst_h.at[pl.ds(o, ch)], idxv)
                pltpu.sync_copy(vals_h[t].at[pl.ds(o, ch)], buf)
                for j in range(ch // 16):
                    tt = idxv[pl.ds(j * 16, 16)] - rbase
                    ok = (tt >= 0) & (tt < nrm)
                    locv[pl.ds(j * 16, 16)] = jnp.where(ok, tt, junk)
                pltpu.sync_copy(buf, acc.at[locv], add=True)
                return carry

            lax.fori_loop(0, nit, step, 0)
            plsc.subcore_barrier()

            o = 0
            while o < rpt:
                sz = min(128, rpt - o)
                pltpu.sync_copy(
                    acc.at[pl.ds(rr0 + o, sz)],
                    outs_h[t].at[pl.ds(rbase + rr0 + o, sz)],
                )
                o += sz

    fn = pl.kernel(
        body,
        out_type=tuple(
            jax.ShapeDtypeStruct((n_pad, 128), F32) for _ in range(t_n)),
        mesh=mesh,
        scratch_types=(
            pltpu.VMEM((ch,), jnp.int32),
            pltpu.VMEM((ch,), jnp.int32),
            pltpu.VMEM((ch, 128), F32),
            pltpu.VMEM_SHARED((rows, 128), F32),
        ),
    )
    outs = fn(*vals_list, dst, zrows)
    if n_pad != n:
        outs = tuple(o[:n] for o in outs)
    return outs


def _sc_scatter_es(vals, dst, n, zrows):
    e = vals.shape[0]
    n_pad = 128 * ((n + 127) // 128)
    eh = e // NC
    et = eh // NS
    ch = _chunk(et)
    nit = et // ch
    mesh = plsc.VectorSubcoreMesh(core_axis_name="c", subcore_axis_name="s")

    def body(vals_h, dst_h, z_h, out_h, idxv, buf, acc):
        cid = lax.axis_index("c")
        sid = lax.axis_index("s")

        rpt = n_pad // NS
        r0 = sid * rpt
        off = 0
        while off < rpt:
            sz = min(128, rpt - off)
            pltpu.sync_copy(z_h.at[pl.ds(0, sz)], acc.at[pl.ds(r0 + off, sz)])
            off += sz
        plsc.subcore_barrier()

        tbase = cid * eh + sid * et

        def step(i, carry):
            o = tbase + i * ch
            pltpu.sync_copy(dst_h.at[pl.ds(o, ch)], idxv)
            pltpu.sync_copy(vals_h.at[pl.ds(o, ch)], buf)
            pltpu.sync_copy(buf, acc.at[idxv], add=True)
            return carry

        lax.fori_loop(0, nit, step, 0)
        plsc.subcore_barrier()

        o = 0
        while o < rpt:
            sz = min(128, rpt - o)
            pltpu.sync_copy(
                acc.at[pl.ds(r0 + o, sz)],
                out_h.at[cid, pl.ds(r0 + o, sz)],
            )
            o += sz

    fn = pl.kernel(
        body,
        out_type=jax.ShapeDtypeStruct((NC, n_pad, 128), F32),
        mesh=mesh,
        scratch_types=(
            pltpu.VMEM((ch,), jnp.int32),
            pltpu.VMEM((ch, 128), F32),
            pltpu.VMEM_SHARED((n_pad, 128), F32),
        ),
    )
    out = fn(vals, dst, zrows)
    return out[:, :n] if n_pad != n else out


def _sc_scatter(vals, dst, n, zrows):
    e = vals.shape[0]
    n_pad = 2048 * ((n + 2047) // 2048)
    nrm = n_pad // 2
    rows = nrm + 128
    junk = nrm
    et = e // NS
    ch = _chunk(et)
    nit = et // ch
    mesh = plsc.VectorSubcoreMesh(core_axis_name="c", subcore_axis_name="s")

    def body(vals_h, dst_h, z_h, out_h, idxv, locv, buf, acc):
        cid = lax.axis_index("c")
        sid = lax.axis_index("s")
        rbase = (cid * nrm).astype(jnp.int32)

        rpt_i = rows // NS
        r0 = sid * rpt_i
        off = 0
        while off < rpt_i:
            sz = min(128, rpt_i - off)
            pltpu.sync_copy(z_h.at[pl.ds(0, sz)], acc.at[pl.ds(r0 + off, sz)])
            off += sz
        plsc.subcore_barrier()

        tbase = sid * et

        def step(i, carry):
            o = tbase + i * ch
            pltpu.sync_copy(dst_h.at[pl.ds(o, ch)], idxv)
            pltpu.sync_copy(vals_h.at[pl.ds(o, ch)], buf)
            for j in range(ch // 16):
                t = idxv[pl.ds(j * 16, 16)] - rbase
                ok = (t >= 0) & (t < nrm)
                locv[pl.ds(j * 16, 16)] = jnp.where(ok, t, junk)
            pltpu.sync_copy(buf, acc.at[locv], add=True)
            return carry

        lax.fori_loop(0, nit, step, 0)
        plsc.subcore_barrier()

        rpt = nrm // NS
        rr0 = sid * rpt
        o = 0
        while o < rpt:
            sz = min(128, rpt - o)
            pltpu.sync_copy(
                acc.at[pl.ds(rr0 + o, sz)],
                out_h.at[pl.ds(rbase + rr0 + o, sz)],
            )
            o += sz

    fn = pl.kernel(
        body,
        out_type=jax.ShapeDtypeStruct((n_pad, 128), F32),
        mesh=mesh,
        scratch_types=(
            pltpu.VMEM((ch,), jnp.int32),
            pltpu.VMEM((ch,), jnp.int32),
            pltpu.VMEM((ch, 128), F32),
            pltpu.VMEM_SHARED((rows, 128), F32),
        ),
    )
    out = fn(vals, dst, zrows)
    return out[:n] if n_pad != n else out


def _fold_bn(g, b, m, v):
    s = g / jnp.sqrt(v + 1e-5)
    return s, b - m * s


def _gated(lp, src, dst, h, p, b3, n, zrows, want_p, want_enx, res=None):
    if want_p:
        wh = jnp.concatenate(
            [lp["B1_W"], lp["B2_W"], lp["A1_W"][:128], lp["A2_W"][:128],
             jnp.zeros((128, 256), F32)], axis=1)
        wp = jnp.concatenate(
            [jnp.zeros((128, 256), F32), lp["A1_W"][128:], lp["A2_W"][128:],
             lp["C1_W"], lp["C2_W"]], axis=1)
        bb = jnp.concatenate(
            [lp["B1_b"], lp["B2_b"], lp["A1_b"], lp["A2_b"],
             lp["C1_b"], lp["C2_b"]])
        b1h, b2h, a1, v, c1, cp = _node_dense(h, p, wh, wp, bb)
    else:
        wh = jnp.concatenate(
            [lp["B1_W"], lp["B2_W"], lp["A1_W"][:128], lp["A2_W"][:128]],
            axis=1)
        wp = jnp.concatenate(
            [jnp.zeros((128, 256), F32), lp["A1_W"][128:], lp["A2_W"][128:]],
            axis=1)
        bb = jnp.concatenate(
            [lp["B1_b"], lp["B2_b"], lp["A1_b"], lp["A2_b"]])
        b1h, b2h, a1, v = _node_dense(h, p, wh, wp, bb)
        c1 = cp = None

    tables = [b2h, v] + ([cp] if want_p else [])
    gathered = _sc_gather(dst, src, b1h, tables)
    bd, bs, vg = gathered[0], gathered[1], gathered[2]
    cg = gathered[3] if want_p else None

    se, te = _fold_bn(lp["bne_g"], lp["bne_b"], lp["bne_m"], lp["bne_v"])
    ef = _edge_fuse(bd, bs, b3, vg, cg, se, te, want_enx)
    sig, ev = ef[0], ef[1]
    i = 2
    ep2 = None
    if want_p:
        ep2 = ef[i]
        i += 1
    enx = ef[i] if want_enx else None

    stacked = n <= 16384
    scat = _sc_scatter_es_multi if stacked else _sc_scatter_multi
    vals = [sig, ev] + ([ep2] if want_p else [])
    sums = scat(vals, dst, n, zrows)
    ssum, sev = sums[0], sums[1]
    sep = sums[2] if want_p else None

    sh, th = _fold_bn(lp["bnh_g"], lp["bnh_b"], lp["bnh_m"], lp["bnh_v"])
    nu = _node_update(ssum, sev, sep, a1, c1, sh, th, res, stacked)
    if want_p:
        return nu[0], nu[1], enx
    return nu[0], None, enx


def kernel(gp_token_res, gp_token_atom, gp_pos_enc, gp_dist, gp_edge_index,
           gl_feat, gl_pos_enc, gl_edge_feat, gl_edge_index,
           gc_dist, gc_edge_index, params):
    pr = params
    n_p = gp_token_res.shape[0]
    n_l = gl_feat.shape[0]
    n_c = n_p + n_l

    zrows = jnp.zeros((128, 128), F32)

    tok = jnp.stack(
        [gp_token_res.astype(jnp.int32), gp_token_atom.astype(jnp.int32)],
        axis=1)
    res_pad = jnp.pad(pr["res_emb"], ((0, 2), (0, 0)))
    atom_pad = jnp.pad(pr["atom_emb"], ((0, 1), (0, 0)))
    hp, pp = _embed_p(tok, gp_pos_enc, res_pad, atom_pad,
                      pr["pp_W"], pr["pp_b"], pr["pnorm_g"], pr["pnorm_b"])
    hl, pl_ = _embed_l(gl_feat, gl_pos_enc, pr["ln_W"], pr["ln_b"],
                       pr["lp_W"], pr["lp_b"], pr["lnorm_g"], pr["lnorm_b"])
    hp_raw, hl_raw = hp, hl
    res_c = jnp.concatenate([hp_raw, hl_raw], axis=0)

    ps, pd = gp_edge_index[0], gp_edge_index[1]
    ls, ld = gl_edge_index[0], gl_edge_index[1]
    cs, cd = gc_edge_index[0], gc_edge_index[1]

    xp = jnp.pad(gp_dist, ((0, 0), (0, 1)))
    xc = jnp.pad(gc_dist, ((0, 0), (0, 1)))
    pe_w = jnp.pad(pr["pe_W"], ((0, 1), (0, 0)))
    ce_w = jnp.pad(pr["ce_W"], ((0, 1), (0, 0)))

    def b3_first(x, ew, eb, blk):
        return _mm(_mm(x, ew, eb), blk["B3_W"], blk["B3_b"])

    enx_p = enx_l = enx_c = None
    hc = None
    for i in range(3):
        bp, bl, bc = pr["pblock"][i], pr["lblock"][i], pr["cblock"][i]
        last = i == 2

        if i == 0:
            b3p = b3_first(xp, pe_w, pr["pe_b"], bp)
            b3l = b3_first(gl_edge_feat, pr["le_W"], pr["le_b"], bl)
        else:
            b3p = _mm(enx_p, bp["B3_W"], bp["B3_b"])
            b3l = _mm(enx_l, bl["B3_W"], bl["B3_b"])
        hp, pp, enx_p = _gated(bp, ps, pd, hp, pp, b3p, n_p, zrows,
                               want_p=True, want_enx=not last)
        hl, pl_, enx_l = _gated(bl, ls, ld, hl, pl_, b3l, n_l, zrows,
                                want_p=True, want_enx=not last)

        hcat = jnp.concatenate([hp, hl], axis=0)
        pcat = jnp.concatenate([pp, pl_], axis=0)
        if i == 0:
            b3c = b3_first(xc, ce_w, pr["ce_b"], bc)
        else:
            b3c = _mm(enx_c, bc["B3_W"], bc["B3_b"])
        hc, _, enx_c = _gated(bc, cs, cd, hcat, pcat, b3c, n_c, zrows,
                              want_p=False, want_enx=not last,
                              res=None if last else res_c)
        if not last:
            hp = hc[:n_p]
            hl = hc[n_p:]

    sm, tm = _fold_bn(pr["mbn_g"], pr["mbn_b"], pr["mbn_m"], pr["mbn_v"])
    return _final(hc, pr["mlp1_W"], pr["mlp1_b"], sm, tm,
                  pr["mlp2_W"], pr["mlp2_b"])
